# baseline (speedup 1.0000x reference)
import functools

import jax
import jax.numpy as jnp
from jax import lax
from jax.experimental import pallas as pl
from jax.experimental.pallas import tpu as pltpu
from jax.experimental.pallas import tpu_sc as plsc

M, D, E = 4096, 1024, 16
TM = 256
NT = 47
NTP = 64
RBUF = NT * TM
RT = 512
NRT = M // RT

NW = 32
TW = M // NW
CD = 32
CC = 16

_NEG = -3.0e38


def _router_body(x_ref, gw_ref, e1_ref, e2_ref, r1_ref, r2_ref, w1_ref,
                 w2_ref, cnt_out_ref, cnt_ref):
    t = pl.program_id(0)

    @pl.when(t == 0)
    def _():
        cnt_ref[...] = jnp.zeros_like(cnt_ref)

    xt = x_ref[...]
    gw = gw_ref[...]
    logits = lax.dot_general(xt, gw, (((1,), (1,)), ((), ())),
                             preferred_element_type=jnp.float32)
    eidx = lax.broadcasted_iota(jnp.int32, (RT, E), 1)
    m1 = jnp.max(logits, axis=1, keepdims=True)
    e1 = jnp.min(jnp.where(logits == m1, eidx, E), axis=1, keepdims=True)
    oh1 = eidx == e1
    l2 = jnp.where(oh1, _NEG, logits)
    m2 = jnp.max(l2, axis=1, keepdims=True)
    e2 = jnp.min(jnp.where(l2 == m2, eidx, E), axis=1, keepdims=True)
    oh2 = eidx == e2
    ew = jnp.exp(m2 - m1)
    w2 = ew / (1.0 + ew)
    w1 = 1.0 - w2

    oh1f = oh1.astype(jnp.float32)
    oh2f = oh2.astype(jnp.float32)
    ohf = oh1f + oh2f
    ri = lax.broadcasted_iota(jnp.int32, (RT, RT), 0)
    ci = lax.broadcasted_iota(jnp.int32, (RT, RT), 1)
    tri = (ri > ci).astype(jnp.float32)
    pre = lax.dot_general(tri, ohf, (((1,), (0,)), ((), ())),
                          preferred_element_type=jnp.float32)
    tot = pre + cnt_ref[...]
    r1 = jnp.sum(oh1f * tot, axis=1, keepdims=True)
    r2 = jnp.sum(oh2f * tot, axis=1, keepdims=True)
    cnt_new = cnt_ref[...] + jnp.sum(ohf, axis=0, keepdims=True)
    cnt_ref[...] = cnt_new

    e1_ref[...] = e1
    e2_ref[...] = e2
    r1_ref[...] = r1.astype(jnp.int32)
    r2_ref[...] = r2.astype(jnp.int32)
    w1_ref[...] = w1
    w2_ref[...] = w2
    cnt_out_ref[...] = jnp.broadcast_to(cnt_new, (8, E)).astype(jnp.int32)


def _router_call(x, gate_weight, interpret=False):
    i32 = jnp.int32
    f32 = jnp.float32
    col = lambda dt: jax.ShapeDtypeStruct((M, 1), dt)
    cspec = lambda: pl.BlockSpec((RT, 1), lambda t: (t, 0))
    fspec = lambda n: pl.BlockSpec((8, n), lambda t: (0, 0))
    return pl.pallas_call(
        _router_body,
        grid=(NRT,),
        in_specs=[
            pl.BlockSpec((RT, D), lambda t: (t, 0)),
            pl.BlockSpec((E, D), lambda t: (0, 0)),
        ],
        out_specs=[cspec(), cspec(), cspec(), cspec(), cspec(), cspec(),
                   fspec(E)],
        out_shape=[col(i32), col(i32), col(i32), col(i32), col(f32), col(f32),
                   jax.ShapeDtypeStruct((8, E), i32)],
        scratch_shapes=[pltpu.VMEM((1, E), f32)],
        interpret=interpret,
    )(x, gate_weight)


def _fixup_body(e1_ref, e2_ref, r1_ref, r2_ref, cnt_ref, d1_ref, d2_ref,
                se_ref, sx_ref, sv_ref):
    cntf = cnt_ref[...].astype(jnp.float32)
    tiles = jnp.floor((cntf + (TM - 1)) * (1.0 / TM))
    lane = lax.broadcasted_iota(jnp.int32, (8, E), 1)
    zero = jnp.zeros((8, E), jnp.float32)
    c = tiles
    for k in (1, 2, 4, 8):
        c = c + jnp.where(lane >= k, pltpu.roll(c, k, axis=1), zero)
    starts = ((c - tiles) * TM).astype(jnp.int32)

    st = starts[0:1, :]
    eidx = lax.broadcasted_iota(jnp.int32, (RT, E), 1)
    stb = jnp.broadcast_to(st, (RT, E))
    zeroi = jnp.zeros((RT, E), jnp.int32)
    d1 = r1_ref[...] + jnp.sum(
        jnp.where(e1_ref[...] == eidx, stb, zeroi), axis=1, keepdims=True)
    d2 = r2_ref[...] + jnp.sum(
        jnp.where(e2_ref[...] == eidx, stb, zeroi), axis=1, keepdims=True)
    d1_ref[...] = d1
    d2_ref[...] = d2

    total_b = jnp.broadcast_to(c[:, E - 1:E], (8, NTP))
    trow = lax.broadcasted_iota(jnp.int32, (8, NTP), 1).astype(jnp.float32)
    sx = jnp.minimum(trow, total_b - 1.0)
    sv = (trow < total_b).astype(jnp.int32)
    ser = jnp.zeros((8, NTP), jnp.int32)
    one = jnp.ones((8, NTP), jnp.int32)
    zero64 = jnp.zeros((8, NTP), jnp.int32)
    for e in range(E):
        cum_e = jnp.broadcast_to(c[:, e:e + 1], (8, NTP))
        ser = ser + jnp.where(sx >= cum_e, one, zero64)
    se_ref[...] = ser
    sx_ref[...] = sx.astype(jnp.int32)
    sv_ref[...] = sv


def _fixup_call(e1o, e2o, r1o, r2o, cnts, interpret=False):
    col = jax.ShapeDtypeStruct((M, 1), jnp.int32)
    cspec = lambda: pl.BlockSpec((RT, 1), lambda t: (t, 0))
    fspec = lambda n: pl.BlockSpec((8, n), lambda t: (0, 0))
    return pl.pallas_call(
        _fixup_body,
        grid=(NRT,),
        in_specs=[cspec(), cspec(), cspec(), cspec(),
                  pl.BlockSpec((8, E), lambda t: (0, 0))],
        out_specs=[cspec(), cspec(), fspec(NTP), fspec(NTP), fspec(NTP)],
        out_shape=[col, col,
                   jax.ShapeDtypeStruct((8, NTP), jnp.int32),
                   jax.ShapeDtypeStruct((8, NTP), jnp.int32),
                   jax.ShapeDtypeStruct((8, NTP), jnp.int32)],
        interpret=interpret,
    )(e1o, e2o, r1o, r2o, cnts)


def _dispatch_body(x_hbm, d1_hbm, d2_hbm, w1_hbm, w2_hbm, xs_hbm, ws_hbm,
                   w_v, dst_v, dstw_v, rows_v, sem_l, sem_s0, sem_s1, semw):
    wid = lax.axis_index("s") * 2 + lax.axis_index("c")
    base = wid * TW
    nck = TW // CD
    ssem = [sem_s0, sem_s1]
    pltpu.sync_copy(w1_hbm.at[pl.ds(base, TW)], w_v.at[0])
    pltpu.sync_copy(w2_hbm.at[pl.ds(base, TW)], w_v.at[1])
    pltpu.sync_copy(d1_hbm.at[pl.ds(base, TW)], dstw_v.at[0])
    pltpu.sync_copy(d2_hbm.at[pl.ds(base, TW)], dstw_v.at[1])
    for s in range(2):
        for g in range(TW // 16):
            dst_v[s * nck + (g * 16) // CD,
                  pl.ds((g * 16) % CD, 16)] = dstw_v[s, pl.ds(g * 16, 16)]
    wsc = (pltpu.async_copy(w_v.at[0], ws_hbm.at[dstw_v.at[0]], semw),
           pltpu.async_copy(w_v.at[1], ws_hbm.at[dstw_v.at[1]], semw))

    loads = {0: pltpu.async_copy(x_hbm.at[pl.ds(base, CD)], rows_v.at[0],
                                 sem_l)}
    scat = {}
    for ck in range(nck):
        p = ck % 2
        loads[ck].wait()
        scat[ck] = (
            pltpu.async_copy(rows_v.at[p], xs_hbm.at[dst_v.at[ck]], ssem[p]),
            pltpu.async_copy(rows_v.at[p], xs_hbm.at[dst_v.at[nck + ck]],
                             ssem[p]),
        )
        if ck + 1 < nck:
            if ck >= 1:
                scat[ck - 1][0].wait()
                scat[ck - 1][1].wait()
            loads[ck + 1] = pltpu.async_copy(
                x_hbm.at[pl.ds(base + (ck + 1) * CD, CD)],
                rows_v.at[(ck + 1) % 2], sem_l)
    scat[nck - 2][0].wait()
    scat[nck - 2][1].wait()
    scat[nck - 1][0].wait()
    scat[nck - 1][1].wait()
    wsc[0].wait()
    wsc[1].wait()


@functools.cache
def _dispatch_kernel():
    return pl.kernel(
        _dispatch_body,
        out_type=[jax.ShapeDtypeStruct((RBUF, D), jnp.float32),
                  jax.ShapeDtypeStruct((RBUF,), jnp.float32)],
        mesh=plsc.VectorSubcoreMesh(core_axis_name="c", subcore_axis_name="s"),
        scratch_types=[
            pltpu.VMEM((2, TW), jnp.float32),
            pltpu.VMEM((2 * TW // CD, CD), jnp.int32),
            pltpu.VMEM((2, TW), jnp.int32),
            pltpu.VMEM((2, CD, D), jnp.float32),
            pltpu.SemaphoreType.DMA,
            pltpu.SemaphoreType.DMA,
            pltpu.SemaphoreType.DMA,
            pltpu.SemaphoreType.DMA,
        ],
    )


def _dispatch_call(*args):
    return _dispatch_kernel()(*args)


def _gmm_body(se_ref, sx_ref, sv_ref, xs_ref, w_ref, ws_ref, y_ref):
    t = pl.program_id(0)

    @pl.when(sv_ref[0, t] == 1)
    def _():
        xw = xs_ref[...] * ws_ref[...]
        y_ref[...] = lax.dot_general(xw, w_ref[0], (((1,), (0,)), ((), ())),
                                     preferred_element_type=jnp.float32)


def _gmm_call(se, sx, sv, xs, expert_weights, ws, interpret=False):
    grid_spec = pltpu.PrefetchScalarGridSpec(
        num_scalar_prefetch=3,
        grid=(NT,),
        in_specs=[
            pl.BlockSpec((TM, D), lambda t, se, sx, sv: (sx[0, t], 0)),
            pl.BlockSpec((1, D, D), lambda t, se, sx, sv: (se[0, t], 0, 0)),
            pl.BlockSpec((TM, 1), lambda t, se, sx, sv: (sx[0, t], 0)),
        ],
        out_specs=pl.BlockSpec((TM, D), lambda t, se, sx, sv: (sx[0, t], 0)),
    )
    return pl.pallas_call(
        _gmm_body,
        grid_spec=grid_spec,
        out_shape=jax.ShapeDtypeStruct((RBUF, D), jnp.float32),
        interpret=interpret,
    )(se, sx, sv, xs, expert_weights, ws)


def _combine_body(y_hbm, d1_hbm, d2_hbm, out_hbm,
                  dst_v, y1_v, y2_v, sem_g0, sem_g1, sem_o):
    wid = lax.axis_index("s") * 2 + lax.axis_index("c")
    base = wid * TW
    nck = TW // CC
    gsem = [sem_g0, sem_g1]
    pltpu.sync_copy(d1_hbm.at[pl.ds(base, TW)], dst_v.at[0])
    pltpu.sync_copy(d2_hbm.at[pl.ds(base, TW)], dst_v.at[1])
    ncg = D // 16

    def start_g(ck):
        p = ck % 2
        return (
            pltpu.async_copy(y_hbm.at[dst_v.at[0, pl.ds(ck * CC, CC)]],
                             y1_v.at[p], gsem[p]),
            pltpu.async_copy(y_hbm.at[dst_v.at[1, pl.ds(ck * CC, CC)]],
                             y2_v.at[p], gsem[p]),
        )

    gath = {0: start_g(0)}
    outs = {}
    for ck in range(nck):
        p = ck % 2
        gath[ck][0].wait()
        gath[ck][1].wait()
        if ck + 1 < nck:
            if ck >= 1:
                outs[ck - 1].wait()
            gath[ck + 1] = start_g(ck + 1)

        def addbody(j, carry):
            r = j // ncg
            c = j % ncg
            sl = pl.ds(c * 16, 16)
            y1_v[p, r, sl] = y1_v[p, r, sl] + y2_v[p, r, sl]
            return carry

        lax.fori_loop(0, CC * ncg, addbody, 0, unroll=4)
        outs[ck] = pltpu.async_copy(
            y1_v.at[p], out_hbm.at[pl.ds(base + ck * CC, CC)], sem_o)
    outs[nck - 2].wait()
    outs[nck - 1].wait()


@functools.cache
def _combine_kernel():
    return pl.kernel(
        _combine_body,
        out_type=jax.ShapeDtypeStruct((M, D), jnp.float32),
        mesh=plsc.VectorSubcoreMesh(core_axis_name="c", subcore_axis_name="s"),
        scratch_types=[
            pltpu.VMEM((2, TW), jnp.int32),
            pltpu.VMEM((2, CC, D), jnp.float32),
            pltpu.VMEM((2, CC, D), jnp.float32),
            pltpu.SemaphoreType.DMA,
            pltpu.SemaphoreType.DMA,
            pltpu.SemaphoreType.DMA,
        ],
    )


def _combine_call(*args):
    return _combine_kernel()(*args)


def kernel(x, gate_weight, expert_weights):
    e1o, e2o, r1o, r2o, w1o, w2o, cnts = _router_call(x, gate_weight)
    d1o, d2o, se, sx, sv = _fixup_call(e1o, e2o, r1o, r2o, cnts)
    d1f = d1o.reshape(M)
    d2f = d2o.reshape(M)

    w1f = w1o.reshape(M)
    w2f = w2o.reshape(M)
    xs, ws = _dispatch_call(x, d1f, d2f, w1f, w2f)
    y = _gmm_call(se, sx, sv, xs, expert_weights, ws.reshape(RBUF, 1))
    out = _combine_call(y, d1f, d2f)
    return out


# speedup vs baseline: 1.0766x; 1.0766x over previous
import functools

import jax
import jax.numpy as jnp
from jax import lax
from jax.experimental import pallas as pl
from jax.experimental.pallas import tpu as pltpu
from jax.experimental.pallas import tpu_sc as plsc

M, D, E = 4096, 1024, 16
TM = 256
NT = 47
NTP = 64
RBUF = NT * TM
RT = 512
NRT = M // RT

NW = 32
TW = M // NW
CD = 32
CC = 16

_NEG = -3.0e38


def _router_body(x_ref, gw_ref, e1_ref, e2_ref, r1_ref, r2_ref, w1_ref,
                 w2_ref, cnt_out_ref, cnt_ref):
    t = pl.program_id(0)

    @pl.when(t == 0)
    def _():
        cnt_ref[...] = jnp.zeros_like(cnt_ref)

    xt = x_ref[...]
    gw = gw_ref[...]
    logits = lax.dot_general(xt, gw, (((1,), (1,)), ((), ())),
                             preferred_element_type=jnp.float32)
    eidx = lax.broadcasted_iota(jnp.int32, (RT, E), 1)
    m1 = jnp.max(logits, axis=1, keepdims=True)
    e1 = jnp.min(jnp.where(logits == m1, eidx, E), axis=1, keepdims=True)
    oh1 = eidx == e1
    l2 = jnp.where(oh1, _NEG, logits)
    m2 = jnp.max(l2, axis=1, keepdims=True)
    e2 = jnp.min(jnp.where(l2 == m2, eidx, E), axis=1, keepdims=True)
    oh2 = eidx == e2
    ew = jnp.exp(m2 - m1)
    w2 = ew / (1.0 + ew)
    w1 = 1.0 - w2

    oh1f = oh1.astype(jnp.float32)
    oh2f = oh2.astype(jnp.float32)
    ohf = oh1f + oh2f
    ri = lax.broadcasted_iota(jnp.int32, (RT, RT), 0)
    ci = lax.broadcasted_iota(jnp.int32, (RT, RT), 1)
    tri = (ri > ci).astype(jnp.float32)
    pre = lax.dot_general(tri, ohf, (((1,), (0,)), ((), ())),
                          preferred_element_type=jnp.float32)
    tot = pre + cnt_ref[...]
    r1 = jnp.sum(oh1f * tot, axis=1, keepdims=True)
    r2 = jnp.sum(oh2f * tot, axis=1, keepdims=True)
    cnt_new = cnt_ref[...] + jnp.sum(ohf, axis=0, keepdims=True)
    cnt_ref[...] = cnt_new

    e1_ref[...] = e1
    e2_ref[...] = e2
    r1_ref[...] = r1.astype(jnp.int32)
    r2_ref[...] = r2.astype(jnp.int32)
    w1_ref[...] = w1
    w2_ref[...] = w2
    cnt_out_ref[...] = jnp.broadcast_to(cnt_new, (8, E)).astype(jnp.int32)


def _router_call(x, gate_weight, interpret=False):
    i32 = jnp.int32
    f32 = jnp.float32
    col = lambda dt: jax.ShapeDtypeStruct((M, 1), dt)
    cspec = lambda: pl.BlockSpec((RT, 1), lambda t: (t, 0))
    fspec = lambda n: pl.BlockSpec((8, n), lambda t: (0, 0))
    return pl.pallas_call(
        _router_body,
        grid=(NRT,),
        in_specs=[
            pl.BlockSpec((RT, D), lambda t: (t, 0)),
            pl.BlockSpec((E, D), lambda t: (0, 0)),
        ],
        out_specs=[cspec(), cspec(), cspec(), cspec(), cspec(), cspec(),
                   fspec(E)],
        out_shape=[col(i32), col(i32), col(i32), col(i32), col(f32), col(f32),
                   jax.ShapeDtypeStruct((8, E), i32)],
        scratch_shapes=[pltpu.VMEM((1, E), f32)],
        interpret=interpret,
    )(x, gate_weight)


def _fixup_body(e1_ref, e2_ref, r1_ref, r2_ref, cnt_ref, d1_ref, d2_ref,
                se_ref, sx_ref, sv_ref):
    cntf = cnt_ref[...].astype(jnp.float32)
    tiles = jnp.floor((cntf + (TM - 1)) * (1.0 / TM))
    lane = lax.broadcasted_iota(jnp.int32, (8, E), 1)
    zero = jnp.zeros((8, E), jnp.float32)
    c = tiles
    for k in (1, 2, 4, 8):
        c = c + jnp.where(lane >= k, pltpu.roll(c, k, axis=1), zero)
    starts = ((c - tiles) * TM).astype(jnp.int32)

    st = starts[0:1, :]
    eidx = lax.broadcasted_iota(jnp.int32, (RT, E), 1)
    stb = jnp.broadcast_to(st, (RT, E))
    zeroi = jnp.zeros((RT, E), jnp.int32)
    d1 = r1_ref[...] + jnp.sum(
        jnp.where(e1_ref[...] == eidx, stb, zeroi), axis=1, keepdims=True)
    d2 = r2_ref[...] + jnp.sum(
        jnp.where(e2_ref[...] == eidx, stb, zeroi), axis=1, keepdims=True)
    d1_ref[...] = d1
    d2_ref[...] = d2

    total_b = jnp.broadcast_to(c[:, E - 1:E], (8, NTP))
    trow = lax.broadcasted_iota(jnp.int32, (8, NTP), 1).astype(jnp.float32)
    sx = jnp.minimum(trow, total_b - 1.0)
    sv = (trow < total_b).astype(jnp.int32)
    ser = jnp.zeros((8, NTP), jnp.int32)
    one = jnp.ones((8, NTP), jnp.int32)
    zero64 = jnp.zeros((8, NTP), jnp.int32)
    for e in range(E):
        cum_e = jnp.broadcast_to(c[:, e:e + 1], (8, NTP))
        ser = ser + jnp.where(sx >= cum_e, one, zero64)
    se_ref[...] = ser
    sx_ref[...] = sx.astype(jnp.int32)
    sv_ref[...] = sv


def _fixup_call(e1o, e2o, r1o, r2o, cnts, interpret=False):
    col = jax.ShapeDtypeStruct((M, 1), jnp.int32)
    cspec = lambda: pl.BlockSpec((RT, 1), lambda t: (t, 0))
    fspec = lambda n: pl.BlockSpec((8, n), lambda t: (0, 0))
    return pl.pallas_call(
        _fixup_body,
        grid=(NRT,),
        in_specs=[cspec(), cspec(), cspec(), cspec(),
                  pl.BlockSpec((8, E), lambda t: (0, 0))],
        out_specs=[cspec(), cspec(), fspec(NTP), fspec(NTP), fspec(NTP)],
        out_shape=[col, col,
                   jax.ShapeDtypeStruct((8, NTP), jnp.int32),
                   jax.ShapeDtypeStruct((8, NTP), jnp.int32),
                   jax.ShapeDtypeStruct((8, NTP), jnp.int32)],
        interpret=interpret,
    )(e1o, e2o, r1o, r2o, cnts)


def _dispatch_body(x_hbm, d1_hbm, d2_hbm, xs_hbm,
                   dst_v, dstw_v, rows_v, sem_l, sem_s0, sem_s1):
    wid = lax.axis_index("s") * 2 + lax.axis_index("c")
    base = wid * TW
    nck = TW // CD
    ssem = [sem_s0, sem_s1]
    pltpu.sync_copy(d1_hbm.at[pl.ds(base, TW)], dstw_v.at[0])
    pltpu.sync_copy(d2_hbm.at[pl.ds(base, TW)], dstw_v.at[1])
    for s in range(2):
        for g in range(TW // 16):
            dst_v[s * nck + (g * 16) // CD,
                  pl.ds((g * 16) % CD, 16)] = dstw_v[s, pl.ds(g * 16, 16)]

    loads = {0: pltpu.async_copy(x_hbm.at[pl.ds(base, CD)], rows_v.at[0],
                                 sem_l)}
    scat = {}
    for ck in range(nck):
        p = ck % 2
        loads[ck].wait()
        scat[ck] = (
            pltpu.async_copy(rows_v.at[p], xs_hbm.at[dst_v.at[ck]], ssem[p]),
            pltpu.async_copy(rows_v.at[p], xs_hbm.at[dst_v.at[nck + ck]],
                             ssem[p]),
        )
        if ck + 1 < nck:
            if ck >= 1:
                scat[ck - 1][0].wait()
                scat[ck - 1][1].wait()
            loads[ck + 1] = pltpu.async_copy(
                x_hbm.at[pl.ds(base + (ck + 1) * CD, CD)],
                rows_v.at[(ck + 1) % 2], sem_l)
    scat[nck - 2][0].wait()
    scat[nck - 2][1].wait()
    scat[nck - 1][0].wait()
    scat[nck - 1][1].wait()


@functools.cache
def _dispatch_kernel():
    return pl.kernel(
        _dispatch_body,
        out_type=jax.ShapeDtypeStruct((RBUF, D), jnp.float32),
        mesh=plsc.VectorSubcoreMesh(core_axis_name="c", subcore_axis_name="s"),
        scratch_types=[
            pltpu.VMEM((2 * TW // CD, CD), jnp.int32),
            pltpu.VMEM((2, TW), jnp.int32),
            pltpu.VMEM((2, CD, D), jnp.float32),
            pltpu.SemaphoreType.DMA,
            pltpu.SemaphoreType.DMA,
            pltpu.SemaphoreType.DMA,
        ],
    )


def _dispatch_call(*args):
    return _dispatch_kernel()(*args)


def _gmm_body(se_ref, sx_ref, sv_ref, xs_ref, w_ref, y_ref):
    t = pl.program_id(0)

    @pl.when(sv_ref[0, t] == 1)
    def _():
        y_ref[...] = lax.dot_general(xs_ref[...], w_ref[0],
                                     (((1,), (0,)), ((), ())),
                                     preferred_element_type=jnp.float32)


def _gmm_call(se, sx, sv, xs, expert_weights, interpret=False):
    grid_spec = pltpu.PrefetchScalarGridSpec(
        num_scalar_prefetch=3,
        grid=(NT,),
        in_specs=[
            pl.BlockSpec((TM, D), lambda t, se, sx, sv: (sx[0, t], 0)),
            pl.BlockSpec((1, D, D), lambda t, se, sx, sv: (se[0, t], 0, 0)),
        ],
        out_specs=pl.BlockSpec((TM, D), lambda t, se, sx, sv: (sx[0, t], 0)),
    )
    return pl.pallas_call(
        _gmm_body,
        grid_spec=grid_spec,
        out_shape=jax.ShapeDtypeStruct((RBUF, D), jnp.float32),
        interpret=interpret,
    )(se, sx, sv, xs, expert_weights)


def _combine_body(y_hbm, d1_hbm, d2_hbm, w1_hbm, w2_hbm, out_hbm,
                  dst_v, w_v, y1_v, y2_v, sem_g0, sem_g1, sem_o):
    wid = lax.axis_index("s") * 2 + lax.axis_index("c")
    base = wid * TW
    nck = TW // CC
    gsem = [sem_g0, sem_g1]
    pltpu.sync_copy(d1_hbm.at[pl.ds(base, TW)], dst_v.at[0])
    pltpu.sync_copy(d2_hbm.at[pl.ds(base, TW)], dst_v.at[1])
    pltpu.sync_copy(w1_hbm.at[pl.ds(base, TW)], w_v.at[0])
    pltpu.sync_copy(w2_hbm.at[pl.ds(base, TW)], w_v.at[1])
    ncg = D // 16

    def start_g(ck):
        p = ck % 2
        return (
            pltpu.async_copy(y_hbm.at[dst_v.at[0, pl.ds(ck * CC, CC)]],
                             y1_v.at[p], gsem[p]),
            pltpu.async_copy(y_hbm.at[dst_v.at[1, pl.ds(ck * CC, CC)]],
                             y2_v.at[p], gsem[p]),
        )

    gath = {0: start_g(0)}
    outs = {}
    for ck in range(nck):
        p = ck % 2
        gath[ck][0].wait()
        gath[ck][1].wait()
        if ck + 1 < nck:
            if ck >= 1:
                outs[ck - 1].wait()
            gath[ck + 1] = start_g(ck + 1)

        w1c = w_v[0, pl.ds(ck * CC, CC)]
        w2c = w_v[1, pl.ds(ck * CC, CC)]
        for r in range(CC):
            idxr = jnp.full((16,), r, jnp.int32)
            s1 = w1c.at[idxr].get(mode="promise_in_bounds")
            s2 = w2c.at[idxr].get(mode="promise_in_bounds")

            def addbody(j, carry, p=p, r=r, s1=s1, s2=s2):
                sl = pl.ds(j * 16, 16)
                y1_v[p, r, sl] = (s1 * y1_v[p, r, sl] +
                                  s2 * y2_v[p, r, sl])
                return carry

            lax.fori_loop(0, ncg, addbody, 0, unroll=4)
        outs[ck] = pltpu.async_copy(
            y1_v.at[p], out_hbm.at[pl.ds(base + ck * CC, CC)], sem_o)
    outs[nck - 2].wait()
    outs[nck - 1].wait()


@functools.cache
def _combine_kernel():
    return pl.kernel(
        _combine_body,
        out_type=jax.ShapeDtypeStruct((M, D), jnp.float32),
        mesh=plsc.VectorSubcoreMesh(core_axis_name="c", subcore_axis_name="s"),
        scratch_types=[
            pltpu.VMEM((2, TW), jnp.int32),
            pltpu.VMEM((2, TW), jnp.float32),
            pltpu.VMEM((2, CC, D), jnp.float32),
            pltpu.VMEM((2, CC, D), jnp.float32),
            pltpu.SemaphoreType.DMA,
            pltpu.SemaphoreType.DMA,
            pltpu.SemaphoreType.DMA,
        ],
    )


def _combine_call(*args):
    return _combine_kernel()(*args)


def kernel(x, gate_weight, expert_weights):
    e1o, e2o, r1o, r2o, w1o, w2o, cnts = _router_call(x, gate_weight)
    d1o, d2o, se, sx, sv = _fixup_call(e1o, e2o, r1o, r2o, cnts)
    d1f = d1o.reshape(M)
    d2f = d2o.reshape(M)

    w1f = w1o.reshape(M)
    w2f = w2o.reshape(M)
    xs = _dispatch_call(x, d1f, d2f)
    y = _gmm_call(se, sx, sv, xs, expert_weights)
    out = _combine_call(y, d1f, d2f, w1f, w2f)
    return out
